# Initial kernel scaffold; baseline (speedup 1.0000x reference)
import jax
import jax.numpy as jnp
from jax import lax
from jax.experimental import pallas as pl
from jax.experimental.pallas import tpu as pltpu
from jax.experimental.pallas import tpu_sc as plsc

_VOCAB = 1000000
_D = 64
_BATCH = 16384
_HIST = 200

_NC = 2
_NS = 16
_NW = _NC * _NS
_BPW = _BATCH // _NW
_CH = 4
_NCH = _BPW // _CH
_LANES = _D // 16


def _body(x_hbm, table_hbm, out_hbm, idx_v, rows_v, out_v, sem):
    wid = lax.axis_index("s") * _NC + lax.axis_index("c")
    base = wid * _BPW

    inv_hist = jnp.float32(1.0 / _HIST)

    def chunk(c, carry):
        s0 = base + c * _CH
        pltpu.sync_copy(x_hbm.at[pl.ds(s0 * _HIST, _CH * _HIST)], idx_v)
        pltpu.async_copy(table_hbm.at[idx_v], rows_v, sem).wait()

        for s in range(_CH):
            r0 = s * _HIST

            def red(j, accs):
                return tuple(
                    accs[k] + rows_v[r0 + j, pl.ds(k * 16, 16)]
                    for k in range(_LANES)
                )

            zero = jnp.zeros((16,), jnp.float32)
            accs = lax.fori_loop(0, _HIST, red, (zero,) * _LANES)
            for k in range(_LANES):
                out_v[s, pl.ds(k * 16, 16)] = accs[k] * inv_hist

        pltpu.sync_copy(out_v, out_hbm.at[pl.ds(s0, _CH)])
        return carry

    lax.fori_loop(0, _NCH, chunk, 0)


@jax.jit
def kernel(x, table):
    xf = x.reshape(-1).astype(jnp.int32)
    mesh = plsc.VectorSubcoreMesh(core_axis_name="c", subcore_axis_name="s")
    f = pl.kernel(
        _body,
        out_type=jax.ShapeDtypeStruct((_BATCH, _D), jnp.float32),
        mesh=mesh,
        scratch_types=[
            pltpu.VMEM((_CH * _HIST,), jnp.int32),
            pltpu.VMEM((_CH * _HIST, _D), jnp.float32),
            pltpu.VMEM((_CH, _D), jnp.float32),
            pltpu.SemaphoreType.DMA,
        ],
    )
    return f(xf, table)


# speedup vs baseline: 2.2818x; 2.2818x over previous
import jax
import jax.numpy as jnp
from jax import lax
from jax.experimental import pallas as pl
from jax.experimental.pallas import tpu as pltpu
from jax.experimental.pallas import tpu_sc as plsc

_VOCAB = 1000000
_D = 64
_BATCH = 16384
_HIST = 200

_NC = 2
_NS = 16
_NW = _NC * _NS
_BPW = _BATCH // _NW
_CH = 4
_NCH = _BPW // _CH
_LANES = _D // 16


def _body(x_hbm, table_hbm, out_hbm, idx_v, rows_v, out_v, sem):
    wid = lax.axis_index("s") * _NC + lax.axis_index("c")
    base = wid * _BPW

    inv_hist = jnp.float32(1.0 / _HIST)

    def chunk(c, carry):
        s0 = base + c * _CH
        pltpu.sync_copy(x_hbm.at[pl.ds(s0 * _HIST, _CH * _HIST)], idx_v)
        pltpu.async_copy(table_hbm.at[idx_v], rows_v, sem).wait()

        for s in range(_CH):
            r0 = s * _HIST

            def red(j, accs):
                return tuple(
                    accs[k] + rows_v[r0 + j, pl.ds(k * 16, 16)]
                    for k in range(_LANES)
                )

            zero = jnp.zeros((16,), jnp.float32)
            accs = lax.fori_loop(0, _HIST, red, (zero,) * _LANES)
            for k in range(_LANES):
                out_v[s, pl.ds(k * 16, 16)] = accs[k] * inv_hist

        pltpu.sync_copy(out_v, out_hbm.at[pl.ds(s0, _CH)])
        return carry

    lax.fori_loop(0, _NCH, chunk, 0)


@jax.jit
def kernel(x, table):
    xf = x.reshape(-1).astype(jnp.int32)
    mesh = plsc.VectorSubcoreMesh(core_axis_name="c", subcore_axis_name="s")
    f = pl.kernel(
        _body,
        out_type=jax.ShapeDtypeStruct((_BATCH, _D), jnp.float32),
        mesh=mesh,
        scratch_types=[
            pltpu.VMEM((_CH * _HIST,), jnp.int32),
            pltpu.VMEM((_CH * _HIST, _D), jnp.float32),
            pltpu.VMEM((_CH, _D), jnp.float32),
            pltpu.SemaphoreType.DMA,
        ],
        compiler_params=pltpu.CompilerParams(use_tc_tiling_on_sc=False),
    )
    return f(xf, table)


# speedup vs baseline: 3.2359x; 1.4182x over previous
import jax
import jax.numpy as jnp
from jax import lax
from jax.experimental import pallas as pl
from jax.experimental.pallas import tpu as pltpu
from jax.experimental.pallas import tpu_sc as plsc

_VOCAB = 1000000
_D = 64
_BATCH = 16384
_HIST = 200

_NC = 2
_NS = 16
_NW = _NC * _NS
_BPW = _BATCH // _NW
_CH = 4
_NCH = _BPW // _CH
_LANES = _D // 16
_UNROLL = 8


def _body(x_hbm, table_hbm, out_hbm, idx_v, rows_v, out_v,
          sem_g0, sem_g1, sem_o0, sem_o1):
    wid = lax.axis_index("s") * _NC + lax.axis_index("c")
    base = wid * _BPW
    sems_g = (sem_g0, sem_g1)
    sems_o = (sem_o0, sem_o1)
    inv_hist = jnp.float32(1.0 / _HIST)

    def load_idx(p, c):
        pltpu.sync_copy(
            x_hbm.at[pl.ds((base + c * _CH) * _HIST, _CH * _HIST)],
            idx_v.at[p])

    def start_gather(p):
        pltpu.async_copy(table_hbm.at[idx_v.at[p]], rows_v.at[p], sems_g[p])

    def slot(c, p):
        pltpu.make_async_copy(
            table_hbm.at[idx_v.at[p]], rows_v.at[p], sems_g[p]).wait()

        @pl.when(c >= 2)
        def _():
            pltpu.make_async_copy(
                out_v.at[p], out_hbm.at[pl.ds(base, _CH)], sems_o[p]).wait()

        for s in range(_CH):
            r0 = s * _HIST

            def red(jj, accs):
                accs = list(accs)
                for u in range(_UNROLL):
                    r = r0 + jj * _UNROLL + u
                    for k in range(_LANES):
                        accs[k] = accs[k] + rows_v[p, r, pl.ds(k * 16, 16)]
                return tuple(accs)

            zero = jnp.zeros((16,), jnp.float32)
            accs = lax.fori_loop(0, _HIST // _UNROLL, red, (zero,) * _LANES)
            for k in range(_LANES):
                out_v[p, s, pl.ds(k * 16, 16)] = accs[k] * inv_hist

        pltpu.async_copy(
            out_v.at[p], out_hbm.at[pl.ds(base + c * _CH, _CH)], sems_o[p])

        @pl.when(c + 2 < _NCH)
        def _():
            load_idx(p, c + 2)
            start_gather(p)

    load_idx(0, 0)
    start_gather(0)
    load_idx(1, 1)
    start_gather(1)

    def step(i, carry):
        slot(2 * i, 0)
        slot(2 * i + 1, 1)
        return carry

    lax.fori_loop(0, _NCH // 2, step, 0)

    for p in (0, 1):
        pltpu.make_async_copy(
            out_v.at[p], out_hbm.at[pl.ds(base, _CH)], sems_o[p]).wait()


@jax.jit
def kernel(x, table):
    xf = x.reshape(-1).astype(jnp.int32)
    mesh = plsc.VectorSubcoreMesh(core_axis_name="c", subcore_axis_name="s")
    f = pl.kernel(
        _body,
        out_type=jax.ShapeDtypeStruct((_BATCH, _D), jnp.float32),
        mesh=mesh,
        scratch_types=[
            pltpu.VMEM((2, _CH * _HIST), jnp.int32),
            pltpu.VMEM((2, _CH * _HIST, _D), jnp.float32),
            pltpu.VMEM((2, _CH, _D), jnp.float32),
            pltpu.SemaphoreType.DMA,
            pltpu.SemaphoreType.DMA,
            pltpu.SemaphoreType.DMA,
            pltpu.SemaphoreType.DMA,
        ],
        compiler_params=pltpu.CompilerParams(use_tc_tiling_on_sc=False),
    )
    return f(xf, table)


# speedup vs baseline: 3.3797x; 1.0444x over previous
import jax
import jax.numpy as jnp
from jax import lax
from jax.experimental import pallas as pl
from jax.experimental.pallas import tpu as pltpu
from jax.experimental.pallas import tpu_sc as plsc

_VOCAB = 1000000
_D = 64
_BATCH = 16384
_HIST = 200

_NC = 2
_NS = 16
_NW = _NC * _NS
_BPW = _BATCH // _NW
_CS = 16
_NCHK = _BPW // _CS
_JB = _HIST // 4
_GROUPS = _NCHK * 4
_NSTEP = _GROUPS // 8
_ROWS = _JB * _CS
_LN = _D // 16


def _body(xT_hbm, table_hbm, out_hbm, idx_v, idx_f, rows_v, out_v,
          si0, si1, si2, si3, sr0, sr1, so0, so1):
    wid = lax.axis_index("s") * _NC + lax.axis_index("c")
    base = wid * _BPW
    sems_i = (si0, si1, si2, si3)
    sems_r = (sr0, sr1)
    sems_o = (so0, so1)
    inv = jnp.float32(1.0 / _HIST)

    def idx_src(c, goff):
        return xT_hbm.at[pl.ds(_JB * goff, _JB), pl.ds(base + c * _CS, _CS)]

    def idx_dma(c, goff, q):
        pltpu.async_copy(idx_src(c, goff), idx_v.at[q], sems_i[q])

    def idx_wait(c, goff, q):
        pltpu.make_async_copy(idx_src(c, goff), idx_v.at[q], sems_i[q]).wait()

    def rearrange(q):
        lanes = lax.iota(jnp.int32, 16) * _JB

        def rb(j, carry):
            plsc.store_scatter(idx_f.at[q], [lanes + j], idx_v[q, j, :])
            return carry

        lax.fori_loop(0, _JB, rb, 0)

    def gather(q, p):
        pltpu.async_copy(table_hbm.at[idx_f.at[q]], rows_v.at[p], sems_r[p])

    def gather_wait(q, p):
        pltpu.make_async_copy(
            table_hbm.at[idx_f.at[q]], rows_v.at[p], sems_r[p]).wait()

    def out_dst(c):
        return out_hbm.at[pl.ds(base + c * _CS, _CS)]

    def out_write(c, k):
        pltpu.async_copy(out_v.at[k], out_dst(c), sems_o[k])

    def out_wait(c, k):
        pltpu.make_async_copy(out_v.at[k], out_dst(c), sems_o[k]).wait()

    def reduce(p, k, goff):
        def per_sample(s, carry):
            if goff == 0:
                accs = (jnp.zeros((16,), jnp.float32),) * _LN
            else:
                accs = tuple(out_v[k, s, pl.ds(t * 16, 16)] for t in range(_LN))

            def rbody(jj, accs):
                accs = list(accs)
                for uu in range(10):
                    r = s * _JB + jj * 10 + uu
                    for t in range(_LN):
                        accs[t] = accs[t] + rows_v[p, r, pl.ds(t * 16, 16)]
                return tuple(accs)

            accs = lax.fori_loop(0, _JB // 10, rbody, accs)
            if goff == 3:
                accs = tuple(a * inv for a in accs)
            for t in range(_LN):
                out_v[k, s, pl.ds(t * 16, 16)] = accs[t]
            return carry

        lax.fori_loop(0, _CS, per_sample, 0)

    def slot(step, u):
        c = 2 * step + (u // 4)
        goff = u % 4
        k = u // 4
        p = u % 2

        gather_wait(goff, p)

        if goff == 0:
            @pl.when(step >= 1)
            def _():
                out_wait(c - 2, k)

        reduce(p, k, goff)

        if goff == 3:
            out_write(c, k)

        g = 8 * step + u
        c2 = 2 * step + (u + 2) // 4
        goff2 = (u + 2) % 4
        q2 = (u + 2) % 4
        c4 = 2 * step + (u + 4) // 4
        goff4 = (u + 4) % 4
        q4 = u % 4

        def issue_gather():
            idx_wait(c2, goff2, q2)
            rearrange(q2)
            gather(q2, p)

        def issue_idx():
            idx_dma(c4, goff4, q4)

        if u < 4:
            issue_gather()
            issue_idx()
        elif u < 6:
            issue_gather()

            @pl.when(step < _NSTEP - 1)
            def _():
                issue_idx()
        else:
            @pl.when(step < _NSTEP - 1)
            def _():
                issue_gather()
                issue_idx()

    for q in range(4):
        idx_dma(0, q, q)
    idx_wait(0, 0, 0)
    rearrange(0)
    gather(0, 0)
    idx_wait(0, 1, 1)
    rearrange(1)
    gather(1, 1)

    def step_body(step, carry):
        for u in range(8):
            slot(step, u)
        return carry

    lax.fori_loop(0, _NSTEP, step_body, 0)

    out_wait(_NCHK - 2, 0)
    out_wait(_NCHK - 1, 1)


@jax.jit
def kernel(x, table):
    xT = x.T.astype(jnp.int32)
    mesh = plsc.VectorSubcoreMesh(core_axis_name="c", subcore_axis_name="s")
    f = pl.kernel(
        _body,
        out_type=jax.ShapeDtypeStruct((_BATCH, _D), jnp.float32),
        mesh=mesh,
        scratch_types=[
            pltpu.VMEM((4, _JB, _CS), jnp.int32),
            pltpu.VMEM((4, _ROWS), jnp.int32),
            pltpu.VMEM((2, _ROWS, _D), jnp.float32),
            pltpu.VMEM((2, _CS, _D), jnp.float32),
            pltpu.SemaphoreType.DMA,
            pltpu.SemaphoreType.DMA,
            pltpu.SemaphoreType.DMA,
            pltpu.SemaphoreType.DMA,
            pltpu.SemaphoreType.DMA,
            pltpu.SemaphoreType.DMA,
            pltpu.SemaphoreType.DMA,
            pltpu.SemaphoreType.DMA,
        ],
        compiler_params=pltpu.CompilerParams(
            use_tc_tiling_on_sc=False, needs_layout_passes=False),
    )
    return f(xT, table)
